# baseline (speedup 1.0000x reference)
import functools

import jax
import jax.numpy as jnp
from jax import lax
from jax.experimental import pallas as pl
from jax.experimental.pallas import tpu as pltpu
from jax.experimental.pallas import tpu_sc as plsc

_B = 4096
_L = 200
_D = 16
_NSP = 26
_SV = 100000

_NW = 32
_SEQ_PW = _B * _L // _NW
_ITEM_PW = _B // _NW
_SP_PW = _B * _NSP // _NW
_CH = 1600
_NCH = _SEQ_PW // _CH

_BB = 32


def _sc_gather_body(seq_tab, sp_tab, seq_idx, item_idx, sp_idx,
                    out_seq, out_item, out_sp,
                    idxv, rowsv, iidxv, irowsv, spidxv, sprowsv, sem):
    nc = 2
    wid = lax.axis_index("s") * nc + lax.axis_index("c")
    base = wid * _SEQ_PW

    pltpu.sync_copy(seq_idx.at[pl.ds(base, _SEQ_PW)], idxv)

    def chunk(j, carry):
        off = j * _CH
        pltpu.async_copy(seq_tab.at[idxv.at[pl.ds(off, _CH)]], rowsv, sem).wait()
        pltpu.sync_copy(rowsv, out_seq.at[pl.ds(base + off, _CH)])
        return carry

    lax.fori_loop(0, _NCH, chunk, 0)

    ibase = wid * _ITEM_PW
    pltpu.sync_copy(item_idx.at[pl.ds(ibase, _ITEM_PW)], iidxv)
    pltpu.async_copy(seq_tab.at[iidxv], irowsv, sem).wait()
    pltpu.sync_copy(irowsv, out_item.at[pl.ds(ibase, _ITEM_PW)])

    sbase = wid * _SP_PW
    pltpu.sync_copy(sp_idx.at[pl.ds(sbase, _SP_PW)], spidxv)
    pltpu.async_copy(sp_tab.at[spidxv], sprowsv, sem).wait()
    pltpu.sync_copy(sprowsv, out_sp.at[pl.ds(sbase, _SP_PW)])


def _sc_gather(seq_table, sp_table_flat, seq_idx_flat, item_idx, sp_idx_flat):
    mesh = plsc.VectorSubcoreMesh(core_axis_name="c", subcore_axis_name="s")
    f = pl.kernel(
        _sc_gather_body,
        out_type=[
            jax.ShapeDtypeStruct((_B * _L, _D), jnp.float32),
            jax.ShapeDtypeStruct((_B, _D), jnp.float32),
            jax.ShapeDtypeStruct((_B * _NSP, _D), jnp.float32),
        ],
        mesh=mesh,
        scratch_types=[
            pltpu.VMEM((_SEQ_PW,), jnp.int32),
            pltpu.VMEM((_CH, _D), jnp.float32),
            pltpu.VMEM((_ITEM_PW,), jnp.int32),
            pltpu.VMEM((_ITEM_PW, _D), jnp.float32),
            pltpu.VMEM((_SP_PW,), jnp.int32),
            pltpu.VMEM((_SP_PW, _D), jnp.float32),
            pltpu.SemaphoreType.DMA,
        ],
        compiler_params=pltpu.CompilerParams(use_tc_tiling_on_sc=False),
    )
    return f(seq_table, sp_table_flat, seq_idx_flat, item_idx, sp_idx_flat)


def _tc_body(seq_ref, sidx_ref, item3_ref, item2_ref, dense_ref, sp_ref,
             w1q_ref, w1s_ref, w1m_ref, b1_ref, w2_ref, b2_ref, wf_ref,
             fw1u_ref, fw1i_ref, fw1d_ref, fw1s_ref, fb1_ref, fw2_ref,
             fb2_ref, ow_ref, ob_ref, out_ref):
    seq2 = seq_ref[:]
    seq3 = seq2.reshape(_BB, _L, _D)
    q3 = item3_ref[:]
    qrep2 = jnp.broadcast_to(q3, (_BB, _L, _D)).reshape(_BB * _L, _D)

    h1 = (seq2 @ w1s_ref[:] + (qrep2 * seq2) @ w1m_ref[:]
          + qrep2 @ w1q_ref[:] + b1_ref[:])
    h1 = jnp.maximum(h1, 0.0)
    h2 = jnp.maximum(h1 @ w2_ref[:] + b2_ref[:], 0.0)
    h23 = h2.reshape(_BB, _L, 40)
    logits3 = jnp.sum(h23 * wf_ref[:], axis=2, keepdims=True)

    mask3 = sidx_ref[:].reshape(_BB, _L, 1) != 0
    lm = jnp.where(mask3, logits3, -1e30)
    m3 = jnp.max(lm, axis=1, keepdims=True)
    e3 = jnp.exp(lm - m3)
    w3 = e3 / jnp.sum(e3, axis=1, keepdims=True)
    user2 = jnp.sum(w3 * seq3, axis=1)

    h = (user2 @ fw1u_ref[:] + item2_ref[:] @ fw1i_ref[:]
         + dense_ref[:] @ fw1d_ref[:] + sp_ref[:] @ fw1s_ref[:] + fb1_ref[:])
    h = jnp.maximum(h, 0.0)
    h = jnp.maximum(h @ fw2_ref[:] + fb2_ref[:], 0.0)
    z = h @ ow_ref[:] + ob_ref[:]
    out_ref[:] = 1.0 / (1.0 + jnp.exp(-z))


def _tc_fused(seq_rows, seq_idx_col, item3, item2, dense, sp2,
              w1q, w1s, w1m, b1, w2, b2, wf3,
              fw1u, fw1i, fw1d, fw1s, fb1, fw2, fb2, ow, ob):
    nblk = _B // _BB

    def bcol(shape):
        return pl.BlockSpec(shape, lambda i: (i,) + (0,) * (len(shape) - 1))

    def full(shape):
        return pl.BlockSpec(shape, lambda i: (0,) * len(shape))

    return pl.pallas_call(
        _tc_body,
        grid=(nblk,),
        in_specs=[
            bcol((_BB * _L, _D)),
            bcol((_BB * _L, 1)),
            bcol((_BB, 1, _D)),
            bcol((_BB, _D)),
            bcol((_BB, 13)),
            bcol((_BB, _NSP * _D)),
            full((_D, 80)), full((_D, 80)), full((_D, 80)), full((1, 80)),
            full((80, 40)), full((1, 40)), full((1, 1, 40)),
            full((_D, 80)), full((_D, 80)), full((13, 80)),
            full((_NSP * _D, 80)), full((1, 80)),
            full((80, 40)), full((1, 40)), full((40, 1)), full((1, 1)),
        ],
        out_specs=bcol((_BB, 1)),
        out_shape=jax.ShapeDtypeStruct((_B, 1), jnp.float32),
        compiler_params=pltpu.CompilerParams(
            dimension_semantics=("arbitrary",),
        ),
    )(seq_rows, seq_idx_col, item3, item2, dense, sp2,
      w1q, w1s, w1m, b1, w2, b2, wf3,
      fw1u, fw1i, fw1d, fw1s, fb1, fw2, fb2, ow, ob)


def kernel(dense_inputs, sparse_inputs, seq_inputs, item_inputs, seq_table,
           sparse_tables, att_W1, att_b1, att_W2, att_b2, att_Wf, att_bf,
           bn_gamma, bn_beta, ffn_W1, ffn_b1, ffn_W2, ffn_b2, out_W, out_b):
    del att_bf
    seq_idx = seq_inputs[:, :, 0]
    seq_idx_flat = seq_idx.reshape(_B * _L)
    item_idx = item_inputs[:, 0]
    sp_idx_flat = (sparse_inputs
                   + jnp.arange(_NSP, dtype=jnp.int32)[None, :] * _SV
                   ).reshape(_B * _NSP)
    sp_table_flat = sparse_tables.reshape(_NSP * _SV, _D)

    seq_rows, item_rows, sp_rows = _sc_gather(
        seq_table, sp_table_flat, seq_idx_flat, item_idx, sp_idx_flat)

    w1q = att_W1[0:_D] + att_W1[2 * _D:3 * _D]
    w1s = att_W1[_D:2 * _D] - att_W1[2 * _D:3 * _D]
    w1m = att_W1[3 * _D:4 * _D]
    b1 = att_b1.reshape(1, 80)
    b2 = att_b2.reshape(1, 40)
    wf3 = att_Wf.reshape(1, 1, 40)
    scale = (bn_gamma * (1.0 / jnp.sqrt(1.0 + 1e-3)))[:, None] * ffn_W1
    fw1u = scale[0:_D]
    fw1i = scale[_D:2 * _D]
    fw1d = scale[2 * _D:2 * _D + 13]
    fw1s = scale[2 * _D + 13:]
    fb1 = (bn_beta @ ffn_W1 + ffn_b1).reshape(1, 80)
    fb2 = ffn_b2.reshape(1, 40)
    ob = out_b.reshape(1, 1)

    return _tc_fused(seq_rows, seq_idx_flat.reshape(_B * _L, 1),
                     item_rows.reshape(_B, 1, _D), item_rows,
                     dense_inputs, sp_rows.reshape(_B, _NSP * _D),
                     w1q, w1s, w1m, b1, att_W2, b2, wf3,
                     fw1u, fw1i, fw1d, fw1s, fb1, ffn_W2, fb2, out_W, ob)


# speedup vs baseline: 1.1885x; 1.1885x over previous
import functools

import jax
import jax.numpy as jnp
from jax import lax
from jax.experimental import pallas as pl
from jax.experimental.pallas import tpu as pltpu
from jax.experimental.pallas import tpu_sc as plsc

_B = 4096
_L = 200
_D = 16
_NSP = 26
_SV = 100000

_NW = 32
_SEQ_PW = _B * _L // _NW
_ITEM_PW = _B // _NW
_SP_PW = _B * _NSP // _NW
_CH = 1600
_NCH = _SEQ_PW // _CH

_BB = 32


def _sc_gather_body(seq_tab, sp_tab, seq_idx, item_idx, sp_idx,
                    out_seq, out_item, out_sp,
                    idxv, rowsv, iidxv, irowsv, spidxv, sprowsv, sem):
    nc = 2
    wid = lax.axis_index("s") * nc + lax.axis_index("c")
    base = wid * _SEQ_PW

    pltpu.sync_copy(seq_idx.at[pl.ds(base, _SEQ_PW)], idxv)

    def chunk(j, carry):
        off = j * _CH
        pltpu.async_copy(seq_tab.at[idxv.at[pl.ds(off, _CH)]], rowsv, sem).wait()
        pltpu.sync_copy(rowsv, out_seq.at[pl.ds(base + off, _CH)])
        return carry

    lax.fori_loop(0, _NCH, chunk, 0)

    ibase = wid * _ITEM_PW
    pltpu.sync_copy(item_idx.at[pl.ds(ibase, _ITEM_PW)], iidxv)
    pltpu.async_copy(seq_tab.at[iidxv], irowsv, sem).wait()
    pltpu.sync_copy(irowsv, out_item.at[pl.ds(ibase, _ITEM_PW)])

    sbase = wid * _SP_PW
    pltpu.sync_copy(sp_idx.at[pl.ds(sbase, _SP_PW)], spidxv)
    pltpu.async_copy(sp_tab.at[spidxv], sprowsv, sem).wait()
    pltpu.sync_copy(sprowsv, out_sp.at[pl.ds(sbase, _SP_PW)])


def _sc_gather(seq_table, sp_table_flat, seq_idx_flat, item_idx, sp_idx_flat):
    mesh = plsc.VectorSubcoreMesh(core_axis_name="c", subcore_axis_name="s")
    f = pl.kernel(
        _sc_gather_body,
        out_type=[
            jax.ShapeDtypeStruct((_B * _L, _D), jnp.float32),
            jax.ShapeDtypeStruct((_B, _D), jnp.float32),
            jax.ShapeDtypeStruct((_B * _NSP, _D), jnp.float32),
        ],
        mesh=mesh,
        scratch_types=[
            pltpu.VMEM((_SEQ_PW,), jnp.int32),
            pltpu.VMEM((_CH, _D), jnp.float32),
            pltpu.VMEM((_ITEM_PW,), jnp.int32),
            pltpu.VMEM((_ITEM_PW, _D), jnp.float32),
            pltpu.VMEM((_SP_PW,), jnp.int32),
            pltpu.VMEM((_SP_PW, _D), jnp.float32),
            pltpu.SemaphoreType.DMA,
        ],
        compiler_params=pltpu.CompilerParams(use_tc_tiling_on_sc=False),
    )
    return f(seq_table, sp_table_flat, seq_idx_flat, item_idx, sp_idx_flat)


def _tc_body(seq_ref, sidx_ref, item2_ref, dense_ref, sp_ref,
             w1q_ref, w1s_ref, w1m_ref, b1_ref, w2_ref, b2_ref, wf_ref,
             fw1u_ref, fw1i_ref, fw1d_ref, fw1s_ref, fb1_ref, fw2_ref,
             fb2_ref, ow_ref, ob_ref, out_ref):
    seq2 = seq_ref[:]
    item2 = item2_ref[:]
    qrep2 = jnp.broadcast_to(item2.reshape(_BB, 1, _D),
                             (_BB, _L, _D)).reshape(_BB * _L, _D)

    qcon = (item2 @ w1q_ref[:] + b1_ref[:]).reshape(_BB, 1, 80)
    h1 = (seq2 @ w1s_ref[:] + (qrep2 * seq2) @ w1m_ref[:]).reshape(_BB, _L, 80)
    h1 = jnp.maximum(h1 + qcon, 0.0).reshape(_BB * _L, 80)
    h2 = jnp.maximum(h1 @ w2_ref[:] + b2_ref[:], 0.0)

    th2 = jnp.transpose(h2.reshape(_BB, _L, 40), (0, 2, 1))
    logt = jnp.sum(th2 * wf_ref[:], axis=1, keepdims=True)

    maskt = sidx_ref[:].reshape(_BB, 1, _L) != 0
    lm = jnp.where(maskt, logt, -1e30)
    m = jnp.max(lm, axis=2, keepdims=True)
    e = jnp.exp(lm - m)
    w = e / jnp.sum(e, axis=2, keepdims=True)
    seqt = jnp.transpose(seq2.reshape(_BB, _L, _D), (0, 2, 1))
    user2 = jnp.sum(w * seqt, axis=2)

    h = (user2 @ fw1u_ref[:] + item2_ref[:] @ fw1i_ref[:]
         + dense_ref[:] @ fw1d_ref[:] + sp_ref[:] @ fw1s_ref[:] + fb1_ref[:])
    h = jnp.maximum(h, 0.0)
    h = jnp.maximum(h @ fw2_ref[:] + fb2_ref[:], 0.0)
    z = h @ ow_ref[:] + ob_ref[:]
    out_ref[:] = 1.0 / (1.0 + jnp.exp(-z))


def _tc_fused(seq_rows, seq_idx2, item2, dense, sp2,
              w1q, w1s, w1m, b1, w2, b2, wf3,
              fw1u, fw1i, fw1d, fw1s, fb1, fw2, fb2, ow, ob):
    nblk = _B // _BB

    def bcol(shape):
        return pl.BlockSpec(shape, lambda i: (i,) + (0,) * (len(shape) - 1))

    def full(shape):
        return pl.BlockSpec(shape, lambda i: (0,) * len(shape))

    return pl.pallas_call(
        _tc_body,
        grid=(nblk,),
        in_specs=[
            bcol((_BB * _L, _D)),
            bcol((_BB, _L)),
            bcol((_BB, _D)),
            bcol((_BB, 13)),
            bcol((_BB, _NSP * _D)),
            full((_D, 80)), full((_D, 80)), full((_D, 80)), full((1, 80)),
            full((80, 40)), full((1, 40)), full((1, 40, 1)),
            full((_D, 80)), full((_D, 80)), full((13, 80)),
            full((_NSP * _D, 80)), full((1, 80)),
            full((80, 40)), full((1, 40)), full((40, 1)), full((1, 1)),
        ],
        out_specs=bcol((_BB, 1)),
        out_shape=jax.ShapeDtypeStruct((_B, 1), jnp.float32),
        compiler_params=pltpu.CompilerParams(
            dimension_semantics=("arbitrary",),
        ),
    )(seq_rows, seq_idx2, item2, dense, sp2,
      w1q, w1s, w1m, b1, w2, b2, wf3,
      fw1u, fw1i, fw1d, fw1s, fb1, fw2, fb2, ow, ob)


def kernel(dense_inputs, sparse_inputs, seq_inputs, item_inputs, seq_table,
           sparse_tables, att_W1, att_b1, att_W2, att_b2, att_Wf, att_bf,
           bn_gamma, bn_beta, ffn_W1, ffn_b1, ffn_W2, ffn_b2, out_W, out_b):
    del att_bf
    seq_idx = seq_inputs[:, :, 0]
    seq_idx_flat = seq_idx.reshape(_B * _L)
    item_idx = item_inputs[:, 0]
    sp_idx_flat = (sparse_inputs
                   + jnp.arange(_NSP, dtype=jnp.int32)[None, :] * _SV
                   ).reshape(_B * _NSP)
    sp_table_flat = sparse_tables.reshape(_NSP * _SV, _D)

    seq_rows, item_rows, sp_rows = _sc_gather(
        seq_table, sp_table_flat, seq_idx_flat, item_idx, sp_idx_flat)

    w1q = att_W1[0:_D] + att_W1[2 * _D:3 * _D]
    w1s = att_W1[_D:2 * _D] - att_W1[2 * _D:3 * _D]
    w1m = att_W1[3 * _D:4 * _D]
    b1 = att_b1.reshape(1, 80)
    b2 = att_b2.reshape(1, 40)
    wf3 = att_Wf.reshape(1, 40, 1)
    scale = (bn_gamma * (1.0 / jnp.sqrt(1.0 + 1e-3)))[:, None] * ffn_W1
    fw1u = scale[0:_D]
    fw1i = scale[_D:2 * _D]
    fw1d = scale[2 * _D:2 * _D + 13]
    fw1s = scale[2 * _D + 13:]
    fb1 = (bn_beta @ ffn_W1 + ffn_b1).reshape(1, 80)
    fb2 = ffn_b2.reshape(1, 40)
    ob = out_b.reshape(1, 1)

    return _tc_fused(seq_rows, seq_idx, item_rows,
                     dense_inputs, sp_rows.reshape(_B, _NSP * _D),
                     w1q, w1s, w1m, b1, att_W2, b2, wf3,
                     fw1u, fw1i, fw1d, fw1s, fb1, ffn_W2, fb2, out_W, ob)


# speedup vs baseline: 1.2002x; 1.0099x over previous
import functools

import jax
import jax.numpy as jnp
from jax import lax
from jax.experimental import pallas as pl
from jax.experimental.pallas import tpu as pltpu
from jax.experimental.pallas import tpu_sc as plsc

_B = 4096
_L = 200
_D = 16
_NSP = 26
_SV = 100000

_NW = 32
_SEQ_PW = _B * _L // _NW
_ITEM_PW = _B // _NW
_SP_PW = _B * _NSP // _NW
_CH = 1600
_NCH = _SEQ_PW // _CH

_BB = 32
_LP = _L // 8
_PR = _BB * _LP


def _sc_gather_body(seq_tab, sp_tab, seq_idx, item_idx, sp_idx,
                    out_seq, out_item, out_sp,
                    idxv, rowsv, iidxv, irowsv, spidxv, sprowsv, sem):
    nc = 2
    wid = lax.axis_index("s") * nc + lax.axis_index("c")
    base = wid * _SEQ_PW

    pltpu.sync_copy(seq_idx.at[pl.ds(base, _SEQ_PW)], idxv)

    def chunk(j, carry):
        off = j * _CH
        pltpu.async_copy(seq_tab.at[idxv.at[pl.ds(off, _CH)]], rowsv, sem).wait()
        pltpu.sync_copy(rowsv, out_seq.at[pl.ds(base + off, _CH)])
        return carry

    lax.fori_loop(0, _NCH, chunk, 0)

    ibase = wid * _ITEM_PW
    pltpu.sync_copy(item_idx.at[pl.ds(ibase, _ITEM_PW)], iidxv)
    pltpu.async_copy(seq_tab.at[iidxv], irowsv, sem).wait()
    pltpu.sync_copy(irowsv, out_item.at[pl.ds(ibase, _ITEM_PW)])

    sbase = wid * _SP_PW
    pltpu.sync_copy(sp_idx.at[pl.ds(sbase, _SP_PW)], spidxv)
    pltpu.async_copy(sp_tab.at[spidxv], sprowsv, sem).wait()
    pltpu.sync_copy(sprowsv, out_sp.at[pl.ds(sbase, _SP_PW)])


def _sc_gather(seq_table, sp_table_flat, seq_idx_flat, item_idx, sp_idx_flat):
    mesh = plsc.VectorSubcoreMesh(core_axis_name="c", subcore_axis_name="s")
    f = pl.kernel(
        _sc_gather_body,
        out_type=[
            jax.ShapeDtypeStruct((_B * _L, _D), jnp.float32),
            jax.ShapeDtypeStruct((_B, _D), jnp.float32),
            jax.ShapeDtypeStruct((_B * _NSP, _D), jnp.float32),
        ],
        mesh=mesh,
        scratch_types=[
            pltpu.VMEM((_SEQ_PW,), jnp.int32),
            pltpu.VMEM((_CH, _D), jnp.float32),
            pltpu.VMEM((_ITEM_PW,), jnp.int32),
            pltpu.VMEM((_ITEM_PW, _D), jnp.float32),
            pltpu.VMEM((_SP_PW,), jnp.int32),
            pltpu.VMEM((_SP_PW, _D), jnp.float32),
            pltpu.SemaphoreType.DMA,
        ],
        compiler_params=pltpu.CompilerParams(use_tc_tiling_on_sc=False),
    )
    return f(seq_table, sp_table_flat, seq_idx_flat, item_idx, sp_idx_flat)


def _tc_body(xp_ref, m128_ref, q128_ref, item2_ref, dense_ref, sp_ref,
             s_ref, w8s_ref, w8m_ref, w1q8_ref, b18_ref, w28_ref, b28_ref,
             wf8_ref, r8_ref, f8_ref,
             fw1u_ref, fw1i_ref, fw1d_ref, fw1s_ref, fb1_ref, fw2_ref,
             fb2_ref, ow_ref, ob_ref, out_ref):
    xp = xp_ref[:]
    item2 = item2_ref[:]
    sel = s_ref[:]
    q128 = sel @ q128_ref[:]

    qcon = item2 @ w1q8_ref[:] + b18_ref[:]
    h1 = xp @ w8s_ref[:] + (q128 * xp) @ w8m_ref[:] + sel @ qcon
    h1 = jnp.maximum(h1, 0.0)
    h2 = jnp.maximum(h1 @ w28_ref[:] + b28_ref[:], 0.0)
    logits8 = (h2 @ wf8_ref[:]).reshape(_BB, _LP, 8)

    m128 = m128_ref[:].reshape(_BB * _L // 128, 128)
    maskp = jnp.stack([m128[:, 8 * s:8 * (s + 1)] for s in range(16)],
                      axis=1).reshape(_BB, _LP, 8)
    lm = jnp.where(maskp > 0.5, logits8, -1e30)
    m = jnp.max(lm, axis=(1, 2), keepdims=True)
    e = jnp.exp(lm - m)
    w = (e / jnp.sum(e, axis=(1, 2), keepdims=True)).reshape(_PR, 8)
    wexp = w @ r8_ref[:]
    t = (wexp * xp).reshape(_BB, _LP, 128)
    user2 = jnp.sum(t, axis=1) @ f8_ref[:]

    h = (user2 @ fw1u_ref[:] + item2_ref[:] @ fw1i_ref[:]
         + dense_ref[:] @ fw1d_ref[:] + sp_ref[:] @ fw1s_ref[:] + fb1_ref[:])
    h = jnp.maximum(h, 0.0)
    h = jnp.maximum(h @ fw2_ref[:] + fb2_ref[:], 0.0)
    z = h @ ow_ref[:] + ob_ref[:]
    out_ref[:] = 1.0 / (1.0 + jnp.exp(-z))


def _tc_fused(xp, m128, q128, item2, dense, sp2,
              sel, w8s, w8m, w1q8, b18, w28, b28, wf8, r8, f8,
              fw1u, fw1i, fw1d, fw1s, fb1, fw2, fb2, ow, ob):
    nblk = _B // _BB

    def bcol(shape):
        return pl.BlockSpec(shape, lambda i: (i,) + (0,) * (len(shape) - 1))

    def full(shape):
        return pl.BlockSpec(shape, lambda i: (0,) * len(shape))

    return pl.pallas_call(
        _tc_body,
        grid=(nblk,),
        in_specs=[
            bcol((_PR, 128)),
            bcol((1, _BB * _L // 128, 128)),
            bcol((_BB, 128)),
            bcol((_BB, _D)),
            bcol((_BB, 13)),
            bcol((_BB, _NSP * _D)),
            full((_PR, _BB)),
            full((128, 640)), full((128, 640)), full((_D, 640)),
            full((1, 640)), full((640, 320)), full((1, 320)),
            full((320, 8)), full((8, 128)), full((128, _D)),
            full((_D, 80)), full((_D, 80)), full((13, 80)),
            full((_NSP * _D, 80)), full((1, 80)),
            full((80, 40)), full((1, 40)), full((40, 1)), full((1, 1)),
        ],
        out_specs=bcol((_BB, 1)),
        out_shape=jax.ShapeDtypeStruct((_B, 1), jnp.float32),
        compiler_params=pltpu.CompilerParams(
            dimension_semantics=("arbitrary",),
        ),
    )(xp, m128, q128, item2, dense, sp2,
      sel, w8s, w8m, w1q8, b18, w28, b28, wf8, r8, f8,
      fw1u, fw1i, fw1d, fw1s, fb1, fw2, fb2, ow, ob)


def kernel(dense_inputs, sparse_inputs, seq_inputs, item_inputs, seq_table,
           sparse_tables, att_W1, att_b1, att_W2, att_b2, att_Wf, att_bf,
           bn_gamma, bn_beta, ffn_W1, ffn_b1, ffn_W2, ffn_b2, out_W, out_b):
    del att_bf
    seq_idx = seq_inputs[:, :, 0]
    seq_idx_flat = seq_idx.reshape(_B * _L)
    item_idx = item_inputs[:, 0]
    sp_idx_flat = (sparse_inputs
                   + jnp.arange(_NSP, dtype=jnp.int32)[None, :] * _SV
                   ).reshape(_B * _NSP)
    sp_table_flat = sparse_tables.reshape(_NSP * _SV, _D)

    seq_rows, item_rows, sp_rows = _sc_gather(
        seq_table, sp_table_flat, seq_idx_flat, item_idx, sp_idx_flat)

    w1q = att_W1[0:_D] + att_W1[2 * _D:3 * _D]
    w1s = att_W1[_D:2 * _D] - att_W1[2 * _D:3 * _D]
    w1m = att_W1[3 * _D:4 * _D]
    eye8 = jnp.eye(8, dtype=jnp.float32)
    w8s = jnp.kron(eye8, w1s)
    w8m = jnp.kron(eye8, w1m)
    w1q8 = jnp.tile(w1q, (1, 8))
    b18 = jnp.tile(att_b1.reshape(1, 80), (1, 8))
    w28 = jnp.kron(eye8, att_W2)
    b28 = jnp.tile(att_b2.reshape(1, 40), (1, 8))
    wf8 = jnp.kron(eye8, att_Wf)
    r8 = jnp.kron(eye8, jnp.ones((1, _D), jnp.float32))
    sel = jnp.kron(jnp.eye(_BB, dtype=jnp.float32),
                   jnp.ones((_LP, 1), jnp.float32))
    f8 = jnp.kron(jnp.ones((8, 1), jnp.float32),
                  jnp.eye(_D, dtype=jnp.float32))
    scale = (bn_gamma * (1.0 / jnp.sqrt(1.0 + 1e-3)))[:, None] * ffn_W1
    fw1u = scale[0:_D]
    fw1i = scale[_D:2 * _D]
    fw1d = scale[2 * _D:2 * _D + 13]
    fw1s = scale[2 * _D + 13:]
    fb1 = (bn_beta @ ffn_W1 + ffn_b1).reshape(1, 80)
    fb2 = ffn_b2.reshape(1, 40)
    ob = out_b.reshape(1, 1)

    xp = seq_rows.reshape(_B * _L * _D // 128, 128)
    m128 = (seq_idx_flat != 0).astype(jnp.float32).reshape(
        _B // _BB, _BB * _L // 128, 128)
    q128 = jnp.tile(item_rows, (1, 8))

    return _tc_fused(xp, m128, q128, item_rows,
                     dense_inputs, sp_rows.reshape(_B, _NSP * _D),
                     sel, w8s, w8m, w1q8, b18, w28, b28, wf8, r8, f8,
                     fw1u, fw1i, fw1d, fw1s, fb1, ffn_W2, fb2, out_W, ob)
